# Initial kernel scaffold; baseline (speedup 1.0000x reference)
import jax
import jax.numpy as jnp
from jax.experimental import pallas as pl


def kernel(x):
    raise NotImplementedError("write your pallas kernel here")


# speedup vs baseline: 3.4864x; 3.4864x over previous
import functools

import jax
import jax.numpy as jnp
from jax import lax
from jax.experimental import pallas as pl
from jax.experimental.pallas import tpu as pltpu
from jax.experimental.pallas import tpu_sc as plsc

ROWS = 128
COLS = 8192
TOPK = 256
LANES = 16
NV = COLS // LANES
NC, NS = 2, 16
NW = NC * NS
ROWS_PER_W = ROWS // NW

MIN32 = -2147483648
POSMASK = 0x7FFFFFFF


def _scan_hist(hist, kneed):
  lane = lax.iota(jnp.int32, LANES)

  def sbody(j, carry):
    found, total, bkt, above = carry
    cj = LANES - 1 - j
    v = hist[pl.ds(cj * LANES, LANES)]
    csum = jnp.sum(v)
    sfx = lax.rev(plsc.cumsum(lax.rev(v, (0,))), (0,))
    condv = (total + sfx) >= kneed
    npc = plsc.all_reduce_population_count(condv)
    jstar = jnp.max(npc) - 1
    fh = jnp.logical_and(found == 0, (total + csum) >= kneed)
    above_here = total + jnp.sum(jnp.where(lane == jstar, sfx - v, 0))
    b_here = cj * LANES + jstar
    bkt = jnp.where(fh, b_here, bkt)
    above = jnp.where(fh, above_here, above)
    found = jnp.where(fh, jnp.int32(1), found)
    return (found, total + csum, bkt, above)

  init = (jnp.int32(0), jnp.int32(0), jnp.int32(0), jnp.int32(0))
  _, _, bkt, above = lax.fori_loop(0, LANES, sbody, init)
  return bkt, above


def _sc_select(xbits):
  mesh = plsc.VectorSubcoreMesh(
      core_axis_name="c", subcore_axis_name="s",
      num_cores=NC, num_subcores=NS)

  @functools.partial(
      pl.kernel,
      out_type=jax.ShapeDtypeStruct((NW, LANES), jnp.int32),
      mesh=mesh,
      compiler_params=pltpu.CompilerParams(needs_layout_passes=False),
      scratch_types=[
          pltpu.VMEM((COLS,), jnp.int32),
          pltpu.VMEM((256,), jnp.int32),
          pltpu.VMEM((LANES,), jnp.int32),
      ],
  )
  def k(x_hbm, thr_hbm, key, hist, thrv):
    wid = lax.axis_index("s") * NC + lax.axis_index("c")
    lane = lax.iota(jnp.int32, LANES)
    ones = jnp.ones((LANES,), jnp.int32)
    zeros16 = jnp.zeros((LANES,), jnp.int32)

    thr_acc = jnp.zeros((LANES,), jnp.int32)
    for r in range(ROWS_PER_W):
      row = wid * ROWS_PER_W + r
      pltpu.sync_copy(x_hbm.at[row], key)

      prefix = jnp.int32(0)
      kneed = jnp.int32(TOPK)
      for level in range(4):
        shift = 24 - 8 * level

        def zbody(i, _):
          hist[pl.ds(i * LANES, LANES)] = zeros16
          return 0
        lax.fori_loop(0, 256 // LANES, zbody, 0)

        if level == 0:
          def pbody(i, _):
            bits = key[pl.ds(i * LANES, LANES)]
            s = jnp.where(bits >= 0, bits, bits ^ POSMASK)
            ub = s ^ MIN32
            key[pl.ds(i * LANES, LANES)] = ub
            idx = lax.shift_right_logical(ub, 24)
            plsc.addupdate_scatter(hist, [idx], ones)
            return 0
        else:
          pfx = prefix

          def pbody(i, _, shift=shift, pfx=pfx):
            ub = key[pl.ds(i * LANES, LANES)]
            m = lax.shift_right_logical(ub, shift + 8) == pfx
            idx = lax.shift_right_logical(ub, shift) & jnp.int32(0xFF)
            plsc.addupdate_scatter(hist, [idx], ones, mask=m)
            return 0
        lax.fori_loop(0, NV, pbody, 0)

        bkt, above = _scan_hist(hist, kneed)
        prefix = lax.shift_left(prefix, 8) | bkt
        kneed = kneed - above

      st = prefix ^ MIN32
      thr_acc = jnp.where(lane == r, st, thr_acc)

    thrv[...] = thr_acc
    pltpu.sync_copy(thrv, thr_hbm.at[wid])

  return k(xbits)


def _tc_finish(x, thr):
  nblk = 16
  bc = COLS // nblk

  def body(x_ref, t_ref, o_ref):
    xv = x_ref[...]
    bits = lax.bitcast_convert_type(xv, jnp.int32)
    s = jnp.where(bits >= 0, bits, bits ^ POSMASK)
    keep = s >= t_ref[...]
    o_ref[...] = jnp.where(keep, jnp.maximum(xv, 0.0), 0.0)

  return pl.pallas_call(
      body,
      grid=(nblk,),
      in_specs=[
          pl.BlockSpec((ROWS, bc), lambda i: (0, i)),
          pl.BlockSpec((ROWS, 1), lambda i: (0, 0)),
      ],
      out_specs=pl.BlockSpec((ROWS, bc), lambda i: (0, i)),
      out_shape=jax.ShapeDtypeStruct((ROWS, COLS), jnp.float32),
  )(x, thr)


def kernel(x):
  xbits = lax.bitcast_convert_type(x, jnp.int32)
  thr = _sc_select(xbits)
  thr = thr[:, :ROWS_PER_W].reshape(ROWS, 1)
  return _tc_finish(x, thr)


# speedup vs baseline: 4.2076x; 1.2069x over previous
import functools

import jax
import jax.numpy as jnp
from jax import lax
from jax.experimental import pallas as pl
from jax.experimental.pallas import tpu as pltpu
from jax.experimental.pallas import tpu_sc as plsc

ROWS = 128
COLS = 8192
TOPK = 256
LANES = 16
NV = COLS // LANES
NC, NS = 2, 16
NW = NC * NS
ROWS_PER_W = ROWS // NW
UNROLL = 4

MIN32 = -2147483648
POSMASK = 0x7FFFFFFF


def _scan_hist(hist, kneed):
  lane = lax.iota(jnp.int32, LANES)

  sums = jnp.zeros((LANES,), jnp.int32)
  for c in range(LANES):
    s = jnp.sum(hist[pl.ds(c * LANES, LANES)])
    sums = jnp.where(lane == c, s, sums)

  sfx = lax.rev(plsc.cumsum(lax.rev(sums, (0,))), (0,))
  condv = sfx >= kneed
  npc = plsc.all_reduce_population_count(condv)
  cstar = jnp.max(npc) - 1
  above_chunks = jnp.sum(jnp.where(lane == cstar, sfx - sums, 0))

  v = hist[pl.ds(cstar * LANES, LANES)]
  sfx2 = lax.rev(plsc.cumsum(lax.rev(v, (0,))), (0,))
  condv2 = (above_chunks + sfx2) >= kneed
  npc2 = plsc.all_reduce_population_count(condv2)
  jstar = jnp.max(npc2) - 1
  above = above_chunks + jnp.sum(jnp.where(lane == jstar, sfx2 - v, 0))
  bkt = cstar * LANES + jstar
  return bkt, above


def kernel(x):
  mesh = plsc.VectorSubcoreMesh(
      core_axis_name="c", subcore_axis_name="s",
      num_cores=NC, num_subcores=NS)

  @functools.partial(
      pl.kernel,
      out_type=jax.ShapeDtypeStruct((ROWS, COLS), jnp.float32),
      mesh=mesh,
      compiler_params=pltpu.CompilerParams(needs_layout_passes=False),
      scratch_types=[
          pltpu.VMEM((COLS,), jnp.float32),
          pltpu.VMEM((COLS,), jnp.int32),
          pltpu.VMEM((256,), jnp.int32),
      ],
  )
  def k(x_hbm, out_hbm, row, key, hist):
    wid = lax.axis_index("s") * NC + lax.axis_index("c")
    ones = jnp.ones((LANES,), jnp.int32)
    zeros16 = jnp.zeros((LANES,), jnp.int32)

    for r in range(ROWS_PER_W):
      row_i = wid * ROWS_PER_W + r
      pltpu.sync_copy(x_hbm.at[row_i], row)

      prefix = jnp.int32(0)
      kneed = jnp.int32(TOPK)
      for level in range(4):
        shift = 24 - 8 * level

        for c in range(256 // LANES):
          hist[pl.ds(c * LANES, LANES)] = zeros16

        if level == 0:
          def pbody(i, _):
            for u in range(UNROLL):
              sl = pl.ds((i * UNROLL + u) * LANES, LANES)
              bits = plsc.bitcast(row[sl], jnp.int32)
              s = jnp.where(bits >= 0, bits, bits ^ POSMASK)
              ub = s ^ MIN32
              key[sl] = ub
              idx = lax.shift_right_logical(ub, 24)
              plsc.addupdate_scatter(hist, [idx], ones)
            return 0
        else:
          pfx = prefix

          def pbody(i, _, shift=shift, pfx=pfx):
            for u in range(UNROLL):
              sl = pl.ds((i * UNROLL + u) * LANES, LANES)
              ub = key[sl]
              m = lax.shift_right_logical(ub, shift + 8) == pfx
              idx = lax.shift_right_logical(ub, shift) & 0xFF
              plsc.addupdate_scatter(hist, [idx], ones, mask=m)
            return 0
        lax.fori_loop(0, NV // UNROLL, pbody, 0)

        bkt, above = _scan_hist(hist, kneed)
        prefix = lax.shift_left(prefix, 8) | bkt
        kneed = kneed - above

      st = prefix ^ MIN32

      def obody(i, _):
        for u in range(UNROLL):
          sl = pl.ds((i * UNROLL + u) * LANES, LANES)
          s = key[sl] ^ MIN32
          keep = jnp.logical_and(s >= st, s > 0)
          outb = jnp.where(keep, s, 0)
          row[sl] = plsc.bitcast(outb, jnp.float32)
        return 0
      lax.fori_loop(0, NV // UNROLL, obody, 0)

      pltpu.sync_copy(row, out_hbm.at[row_i])

  return k(x)
